# Initial kernel scaffold; baseline (speedup 1.0000x reference)
import jax
import jax.numpy as jnp
from jax.experimental import pallas as pl


def kernel(x, edge_index, edge_values, Ws, bs, Wn, bn, Wg1, bg1, Wg2, bg2):
    raise NotImplementedError("write your pallas kernel here")


# speedup vs baseline: 3.1006x; 3.1006x over previous
import functools

import jax
import jax.numpy as jnp
from jax import lax
from jax.experimental import pallas as pl
from jax.experimental.pallas import tpu as pltpu
from jax.experimental.pallas import tpu_sc as plsc

N = 10000
C = 256
E = 160000
CH = 128
BN = 1000
K = 128
NCHUNKS = E // K
NSUB = 16
ROWS_PER_SUB = 632
NPAD = NSUB * ROWS_PER_SUB

_PREC = lax.Precision.HIGHEST


def _dotT(a, b):
    return lax.dot_general(a, b, (((1,), (1,)), ((), ())),
                           preferred_element_type=jnp.float32,
                           precision=_PREC)


def _nf_body(x_ref, wn_ref, bn_ref, nf_ref):
    nf_ref[...] = _dotT(x_ref[...], wn_ref[...]) + bn_ref[...]


def _nf_call(x, Wn, bn2):
    return pl.pallas_call(
        _nf_body,
        grid=(2, N // BN),
        in_specs=[
            pl.BlockSpec((BN, C), lambda h, i: (i, 0)),
            pl.BlockSpec((CH, C), lambda h, i: (h, 0)),
            pl.BlockSpec((1, CH), lambda h, i: (0, h)),
        ],
        out_specs=pl.BlockSpec((BN, CH), lambda h, i: (h * (N // BN) + i, 0)),
        out_shape=jax.ShapeDtypeStruct((2 * N, CH), jnp.float32),
    )(x, Wn, bn2)


def _t_body(x_ref, ws_ref, bs_ref, wg1a_ref, t_ref):
    s = _dotT(x_ref[...], ws_ref[...]) + bs_ref[...]
    t_ref[...] = _dotT(s, wg1a_ref[...])


def _t_call(x, Ws, bs2, Wg1a):
    return pl.pallas_call(
        _t_body,
        grid=(N // BN,),
        in_specs=[
            pl.BlockSpec((BN, C), lambda i: (i, 0)),
            pl.BlockSpec((C, C), lambda i: (0, 0)),
            pl.BlockSpec((1, C), lambda i: (0, 0)),
            pl.BlockSpec((C, C), lambda i: (0, 0)),
        ],
        out_specs=pl.BlockSpec((BN, C), lambda i: (i, 0)),
        out_shape=jax.ShapeDtypeStruct((N, C), jnp.float32),
    )(x, Ws, bs2, Wg1a)


def _sc_aggr_body(nf_hbm, row_hbm, col_hbm, w_hbm, z_hbm, out_hbm,
                  aggr_sh, colv, rowv, wv, msgs, gsem):
    cidx = lax.axis_index("c")
    sidx = lax.axis_index("s")

    pltpu.sync_copy(z_hbm, aggr_sh.at[pl.ds(sidx * ROWS_PER_SUB, ROWS_PER_SUB)])
    plsc.subcore_barrier()

    col_off = cidx * N

    @pl.loop(0, (NCHUNKS + NSUB - 1) // NSUB)
    def _chunks(i):
        q = sidx + i * NSUB

        @pl.when(q < NCHUNKS)
        def _():
            base = q * K
            pltpu.sync_copy(col_hbm.at[pl.ds(base, K)], colv)
            pltpu.sync_copy(row_hbm.at[pl.ds(base, K)], rowv)
            pltpu.sync_copy(w_hbm.at[pl.ds(base, K)], wv)

            @pl.loop(0, K // 16)
            def _adj(g):
                sl = pl.ds(g * 16, 16)
                colv[sl] = colv[sl] + col_off

            pltpu.async_copy(nf_hbm.at[colv], msgs, gsem).wait()

            @pl.loop(0, K // 16)
            def _scale(g):
                wvec = wv[pl.ds(g * 16, 16)]
                for j in range(16):
                    wb = wvec[jnp.full((16,), j, jnp.int32)]
                    e = g * 16 + j
                    for h in range(CH // 16):
                        sl = pl.ds(h * 16, 16)
                        msgs[e, sl] = msgs[e, sl] * wb

            pltpu.sync_copy(msgs, aggr_sh.at[rowv], add=True)

    plsc.subcore_barrier()
    pltpu.sync_copy(aggr_sh.at[pl.ds(sidx * ROWS_PER_SUB, ROWS_PER_SUB)],
                    out_hbm.at[cidx, pl.ds(sidx * ROWS_PER_SUB, ROWS_PER_SUB)])


def _sc_aggr(nf_cat, row, col, w, zeros):
    mesh = plsc.VectorSubcoreMesh(core_axis_name="c", subcore_axis_name="s")
    kern = pl.kernel(
        _sc_aggr_body,
        out_type=jax.ShapeDtypeStruct((2, NPAD, CH), jnp.float32),
        mesh=mesh,
        scratch_types=[
            pltpu.VMEM_SHARED((NPAD, CH), jnp.float32),
            pltpu.VMEM((K,), jnp.int32),
            pltpu.VMEM((K,), jnp.int32),
            pltpu.VMEM((K,), jnp.float32),
            pltpu.VMEM((K, CH), jnp.float32),
            pltpu.SemaphoreType.DMA,
        ],
    )
    return kern(nf_cat, row, col, w, zeros)


def _b_body(x_ref, t_ref, a0_ref, a1_ref, wg1b0_ref, wg1b1_ref, bg1_ref,
            wg2_ref, bg2_ref, out_ref):
    gp = (t_ref[...] + _dotT(a0_ref[0], wg1b0_ref[...])
          + _dotT(a1_ref[0], wg1b1_ref[...]) + bg1_ref[...])
    g = 0.5 * gp * (1.0 + lax.erf(gp * 0.7071067811865476))
    out_ref[...] = x_ref[...] + _dotT(g, wg2_ref[...]) + bg2_ref[...]


def _b_call(x, t, a_cat, Wg1b0, Wg1b1, bg12, Wg2, bg22):
    return pl.pallas_call(
        _b_body,
        grid=(N // BN,),
        in_specs=[
            pl.BlockSpec((BN, C), lambda i: (i, 0)),
            pl.BlockSpec((BN, C), lambda i: (i, 0)),
            pl.BlockSpec((1, BN, CH), lambda i: (0, i, 0)),
            pl.BlockSpec((1, BN, CH), lambda i: (1, i, 0)),
            pl.BlockSpec((C, CH), lambda i: (0, 0)),
            pl.BlockSpec((C, CH), lambda i: (0, 0)),
            pl.BlockSpec((1, C), lambda i: (0, 0)),
            pl.BlockSpec((C, C), lambda i: (0, 0)),
            pl.BlockSpec((1, C), lambda i: (0, 0)),
        ],
        out_specs=pl.BlockSpec((BN, C), lambda i: (i, 0)),
        out_shape=jax.ShapeDtypeStruct((N, C), jnp.float32),
    )(x, t, a_cat, a_cat, Wg1b0, Wg1b1, bg12, Wg2, bg22)


def kernel(x, edge_index, edge_values, Ws, bs, Wn, bn, Wg1, bg1, Wg2, bg2):
    x_flat = x[0]
    row = edge_index[0].astype(jnp.int32)
    col = edge_index[1].astype(jnp.int32)
    w = edge_values.astype(jnp.float32)
    zeros = jnp.zeros((ROWS_PER_SUB, CH), jnp.float32)

    nf_cat = _nf_call(x_flat, Wn, bn.reshape(1, C))
    t = _t_call(x_flat, Ws, bs.reshape(1, C), Wg1[:, :C])
    a_cat = _sc_aggr(nf_cat, row, col, w, zeros)
    out = _b_call(x_flat, t, a_cat, Wg1[:, C:C + CH], Wg1[:, C + CH:],
                  bg1.reshape(1, C), Wg2, bg2.reshape(1, C))
    return out[None]
